# Initial kernel scaffold; baseline (speedup 1.0000x reference)
import jax
import jax.numpy as jnp
from jax.experimental import pallas as pl


def kernel(x, edge_index, edge_weight, Wz, bz, Wlz, blz, Wr, br, Wlr, blr, Wh, bh, Wlh, blh, Wout, bout):
    raise NotImplementedError("write your pallas kernel here")


# speedup vs baseline: 22.0342x; 22.0342x over previous
import functools
import jax
import jax.numpy as jnp
from jax import lax
from jax.experimental import pallas as pl
from jax.experimental.pallas import tpu as pltpu
from jax.experimental.pallas import tpu_sc as plsc

F32 = jnp.float32


def _newton_rsqrt(d):
    i = lax.bitcast_convert_type(d, jnp.int32)
    i = jnp.int32(0x5F3759DF) - (i >> 1)
    y = lax.bitcast_convert_type(i, F32)
    for _ in range(3):
        y = y * (1.5 - 0.5 * d * y * y)
    return jnp.where(d > 0.0, y, 0.0)


def _make_sc_propagate(N, E, F):
    assert F == 128 and N % 16 == 0 and E % 256 == 0
    EROWS = E // 128
    RB1 = 10
    assert EROWS % RB1 == 0
    NB1 = EROWS // RB1
    EPC = EROWS // 2
    R2 = EPC // 16
    R2X = EPC % 16
    CH = 640
    LAST = N - 15 * CH
    assert 0 < LAST <= CH and LAST % 8 == 0 and CH % 128 == 0
    LASTF = LAST // 32
    LASTR = LAST % 32

    mesh = plsc.VectorSubcoreMesh(core_axis_name="c", subcore_axis_name="s")

    @functools.partial(
        pl.kernel,
        mesh=mesh,
        compiler_params=pltpu.CompilerParams(needs_layout_passes=False),
        out_type=[
            jax.ShapeDtypeStruct((2 * N, F), F32),
            jax.ShapeDtypeStruct((N,), F32),
        ],
        scratch_types=[
            pltpu.VMEM_SHARED((N, F), F32),
            pltpu.VMEM_SHARED((N,), F32),
            pltpu.VMEM((N,), F32),
            pltpu.VMEM((RB1, 128), jnp.int32),
            pltpu.VMEM((RB1 * 128,), F32),
            pltpu.VMEM((128,), jnp.int32),
            pltpu.VMEM((128, F), F32),
            pltpu.VMEM((128,), F32),
            pltpu.VMEM((32, F), F32),
            pltpu.VMEM((CH,), F32),
            pltpu.SemaphoreType.DMA,
            pltpu.SemaphoreType.DMA,
        ],
    )
    def sc_fn(src_hbm, dst_hbm, w_hbm, x_hbm, pout_hbm, dinv_hbm,
              p_sh, deg_sh, dinv_v, dstb, wb, srcb, rows, normb, zbuf,
              onesb, sem_s, sem_g):
        c = lax.axis_index("c")
        s = lax.axis_index("s")

        z16 = jnp.zeros((16,), F32)
        o16 = jnp.ones((16,), F32)

        def fill_z(i, _):
            for j in range(F // 16):
                zbuf[i, pl.ds(16 * j, 16)] = z16
            return 0
        lax.fori_loop(0, 32, fill_z, 0)

        def fill_o(i, _):
            onesb[pl.ds(16 * i, 16)] = o16
            return 0
        lax.fori_loop(0, CH // 16, fill_o, 0)

        @pl.when(s < 15)
        def _():
            for k in range(CH // 32):
                pltpu.sync_copy(zbuf, p_sh.at[pl.ds(s * CH + k * 32, 32)])
            pltpu.sync_copy(onesb, deg_sh.at[pl.ds(s * CH, CH)])

        @pl.when(s == 15)
        def _():
            for k in range(LASTF):
                pltpu.sync_copy(zbuf, p_sh.at[pl.ds(15 * CH + k * 32, 32)])
            if LASTR:
                pltpu.sync_copy(
                    zbuf.at[pl.ds(0, LASTR)],
                    p_sh.at[pl.ds(15 * CH + LASTF * 32, LASTR)])
            pltpu.sync_copy(onesb.at[pl.ds(0, LAST)],
                            deg_sh.at[pl.ds(15 * CH, LAST)])

        plsc.subcore_barrier()

        n1 = (NB1 // 16) + jnp.where(s < (NB1 % 16), 1, 0)

        def p1(k, _):
            e0 = (s + 16 * k) * RB1 * 128
            pltpu.sync_copy(w_hbm.at[pl.ds(e0, RB1 * 128)], wb)
            for j in range(RB1):
                pltpu.sync_copy(dst_hbm.at[pl.ds(e0 + 128 * j, 128)],
                                dstb.at[j])
            descs = [
                pltpu.async_copy(wb.at[pl.ds(128 * j, 128)],
                                 deg_sh.at[dstb.at[j]], sem_s, add=True)
                for j in range(RB1)
            ]
            for dsc in descs:
                dsc.wait()
            return 0

        lax.fori_loop(0, n1, p1, 0)
        plsc.subcore_barrier()

        pltpu.sync_copy(deg_sh, dinv_v)

        def inv(i, _):
            d = dinv_v[pl.ds(16 * i, 16)]
            dinv_v[pl.ds(16 * i, 16)] = _newton_rsqrt(d)
            return 0
        lax.fori_loop(0, N // 16, inv, 0)
        plsc.subcore_barrier()

        r0 = c * EPC + s * R2 + jnp.minimum(s, R2X)
        n2 = R2 + jnp.where(s < R2X, 1, 0)

        def p2(k, _):
            e0 = (r0 + k) * 128
            pltpu.sync_copy(src_hbm.at[pl.ds(e0, 128)], srcb)
            pltpu.sync_copy(dst_hbm.at[pl.ds(e0, 128)], dstb.at[0])
            pltpu.sync_copy(w_hbm.at[pl.ds(e0, 128)],
                            wb.at[pl.ds(0, 128)])
            pltpu.async_copy(x_hbm.at[srcb], rows, sem_g).wait()
            for j in range(8):
                sl = pl.ds(16 * j, 16)
                s16 = srcb[sl]
                d16 = dstb[0, sl]
                w16 = wb[sl]
                n16 = (plsc.load_gather(dinv_v, [s16]) * w16
                       * plsc.load_gather(dinv_v, [d16]))
                normb[sl] = n16

            def scale(e, _):
                spl = plsc.load_gather(normb, [jnp.full((16,), e, jnp.int32)])
                for cb in range(F // 16):
                    sl = pl.ds(16 * cb, 16)
                    rows[e, sl] = rows[e, sl] * spl
                return 0
            lax.fori_loop(0, 128, scale, 0)
            pltpu.async_copy(rows, p_sh.at[dstb.at[0]], sem_s,
                             add=True).wait()
            return 0

        lax.fori_loop(0, n2, p2, 0)
        plsc.subcore_barrier()

        @pl.when(s < 15)
        def _():
            pltpu.sync_copy(p_sh.at[pl.ds(s * CH, CH)],
                            pout_hbm.at[pl.ds(c * N + s * CH, CH)])

        @pl.when(s == 15)
        def _():
            pltpu.sync_copy(p_sh.at[pl.ds(15 * CH, LAST)],
                            pout_hbm.at[pl.ds(c * N + 15 * CH, LAST)])

        @pl.when(jnp.logical_and(c == 0, s == 0))
        def _():
            pltpu.sync_copy(dinv_v, dinv_hbm)

    return sc_fn


def _tc_epilogue(t0, t1, x, dinv2, Wz, bz2, Wlzt, blz2, Wh, bh2, Wlht,
                 blh2, Wout, bout2):
    N, F = x.shape
    BN = 1000
    assert N % BN == 0

    def body(t0_r, t1_r, x_r, di_r, wz_r, bz_r, wlz_r, blz_r, wh_r, bh_r,
             wlh_r, blh_r, wo_r, bo_r, o_r):
        di = di_r[...]
        P = t0_r[...] + t1_r[...] + di * di * x_r[...]
        wzf = jnp.dot(wz_r[...], wlz_r[...], preferred_element_type=F32)
        bzf = jnp.dot(bz_r[...], wlz_r[...], preferred_element_type=F32) \
            + blz_r[...]
        whf = jnp.dot(wh_r[...], wlh_r[...], preferred_element_type=F32)
        bhf = jnp.dot(bh_r[...], wlh_r[...], preferred_element_type=F32) \
            + blh_r[...]
        Z = jax.nn.sigmoid(jnp.dot(P, wzf, preferred_element_type=F32) + bzf)
        Ht = jnp.tanh(jnp.dot(P, whf, preferred_element_type=F32) + bhf)
        H = (1.0 - Z) * Ht
        o_r[...] = jnp.dot(jnp.maximum(H, 0.0), wo_r[...],
                           preferred_element_type=F32) + bo_r[...]

    full = lambda shape: pl.BlockSpec(shape, lambda i: (0, 0))
    rowb = lambda: pl.BlockSpec((BN, F), lambda i: (i, 0))
    return pl.pallas_call(
        body,
        grid=(N // BN,),
        in_specs=[
            rowb(), rowb(), rowb(),
            pl.BlockSpec((BN, 1), lambda i: (i, 0)),
            full((F, F)), full((1, F)), full((F, F)), full((1, F)),
            full((F, F)), full((1, F)), full((F, F)), full((1, F)),
            full((F, 1)), full((1, 1)),
        ],
        out_specs=pl.BlockSpec((BN, 1), lambda i: (i, 0)),
        out_shape=jax.ShapeDtypeStruct((N, 1), F32),
    )(t0, t1, x, dinv2, Wz, bz2, Wlzt, blz2, Wh, bh2, Wlht, blh2, Wout,
      bout2)


@jax.jit
def kernel(x, edge_index, edge_weight, Wz, bz, Wlz, blz, Wr, br, Wlr, blr,
           Wh, bh, Wlh, blh, Wout, bout):
    N, F = x.shape
    E = edge_index.shape[1]
    sc_fn = _make_sc_propagate(N, E, F)
    pout, dinv = sc_fn(edge_index[0], edge_index[1], edge_weight, x)

    return _tc_epilogue(
        pout[:N], pout[N:], x, dinv[:, None],
        Wz, bz[None, :], Wlz[:F], blz[None, :],
        Wh, bh[None, :], Wlh[:F], blh[None, :],
        Wout, bout[None, :],
    )
